# Initial kernel scaffold; baseline (speedup 1.0000x reference)
import jax
import jax.numpy as jnp
from jax.experimental import pallas as pl


def kernel(vecs, codewords):
    raise NotImplementedError("write your pallas kernel here")


# speedup vs baseline: 11.2717x; 11.2717x over previous
import functools

import jax
import jax.numpy as jnp
from jax import lax
from jax.experimental import pallas as pl
from jax.experimental.pallas import tpu as pltpu
from jax.experimental.pallas import tpu_sc as plsc

M = 8
KS = 256
DS = 16

NC = 2
NS = 16
NW = NC * NS


def _encode_body(vecs_ref, cwt_ref, codes_ref):
    b = vecs_ref.shape[0]
    for m in range(M):
        sub = vecs_ref[:, m * DS:(m + 1) * DS]
        cwt = cwt_ref[m * DS:(m + 1) * DS, :]
        xc = jnp.dot(sub, cwt, preferred_element_type=jnp.float32)
        c2 = jnp.sum(cwt * cwt, axis=0, keepdims=True)
        score = c2 - 2.0 * xc
        minval = jnp.min(score, axis=1, keepdims=True)
        lane = lax.broadcasted_iota(jnp.int32, (b, KS), 1)
        idx = jnp.min(jnp.where(score == minval, lane, KS), axis=1)
        codes_ref[m, :] = idx + m * KS


def _encode(vecs, cwt, block_b):
    n = vecs.shape[0]
    grid = (n // block_b,)
    return pl.pallas_call(
        _encode_body,
        grid=grid,
        in_specs=[
            pl.BlockSpec((block_b, M * DS), lambda i: (i, 0)),
            pl.BlockSpec((M * DS, KS), lambda i: (0, 0)),
        ],
        out_specs=pl.BlockSpec((M, block_b), lambda i: (0, i)),
        out_shape=jax.ShapeDtypeStruct((M, n), jnp.int32),
    )(vecs, cwt)


def _make_decode(total, chunk):
    b_per_w = total // NW
    n_chunks = b_per_w // chunk
    mesh = plsc.VectorSubcoreMesh(
        core_axis_name="c", subcore_axis_name="s",
        num_cores=NC, num_subcores=NS)

    @functools.partial(
        pl.kernel,
        out_type=jax.ShapeDtypeStruct((total, DS), jnp.float32),
        mesh=mesh,
        scratch_types=[
            pltpu.VMEM((chunk,), jnp.int32),
            pltpu.VMEM((chunk, DS), jnp.float32),
            pltpu.SemaphoreType.DMA,
        ],
        compiler_params=pltpu.CompilerParams(use_tc_tiling_on_sc=False),
    )
    def decode(table_hbm, idx_hbm, out_hbm, idx_v, rows_v, sem):
        wid = lax.axis_index("s") * NC + lax.axis_index("c")
        base = wid * b_per_w
        for c in range(n_chunks):
            off = base + c * chunk
            pltpu.sync_copy(idx_hbm.at[pl.ds(off, chunk)], idx_v)
            pltpu.async_copy(table_hbm.at[idx_v], rows_v, sem).wait()
            pltpu.sync_copy(rows_v, out_hbm.at[pl.ds(off, chunk)])

    return decode


def kernel(vecs, codewords):
    n, d = vecs.shape
    m_, ks_, ds_ = codewords.shape
    cwt = codewords.transpose(0, 2, 1).reshape(m_ * ds_, ks_)
    codes = _encode(vecs, cwt, block_b=1024)
    flat_codes = codes.T.reshape(n * m_)
    table = codewords.reshape(m_ * ks_, ds_)
    rows = _make_decode(n * m_, 2048)(table, flat_codes)
    return rows.reshape(n, d)


# speedup vs baseline: 17.0039x; 1.5085x over previous
import functools

import jax
import jax.numpy as jnp
from jax import lax
from jax.experimental import pallas as pl
from jax.experimental.pallas import tpu as pltpu
from jax.experimental.pallas import tpu_sc as plsc

M = 8
KS = 256
DS = 16

NC = 2
NS = 16
NW = NC * NS


def _encode_body(vecs_ref, cwt_ref, codes_ref):
    b = vecs_ref.shape[0]
    cols = []
    for m in range(M):
        sub = vecs_ref[:, m * DS:(m + 1) * DS]
        cwt = cwt_ref[m * DS:(m + 1) * DS, :]
        xc = jnp.dot(sub, cwt, preferred_element_type=jnp.float32)
        c2h = 0.5 * jnp.sum(cwt * cwt, axis=0, keepdims=True)
        score = c2h - xc
        minval = jnp.min(score, axis=1, keepdims=True)
        lane = lax.broadcasted_iota(jnp.int32, (b, KS), 1).astype(jnp.float32)
        idx = jnp.min(jnp.where(score == minval, lane, float(KS)),
                      axis=1, keepdims=True)
        cols.append(idx + m * KS)
    codes_ref[:, :] = jnp.concatenate(cols, axis=1).astype(jnp.int32)


def _encode(vecs, cwt, block_b):
    n = vecs.shape[0]
    grid = (n // block_b,)
    return pl.pallas_call(
        _encode_body,
        grid=grid,
        in_specs=[
            pl.BlockSpec((block_b, M * DS), lambda i: (i, 0)),
            pl.BlockSpec((M * DS, KS), lambda i: (0, 0)),
        ],
        out_specs=pl.BlockSpec((block_b, M), lambda i: (i, 0)),
        out_shape=jax.ShapeDtypeStruct((n, M), jnp.int32),
    )(vecs, cwt)


def _make_decode(total, chunk):
    b_per_w = total // NW
    n_chunks = b_per_w // chunk
    mesh = plsc.VectorSubcoreMesh(
        core_axis_name="c", subcore_axis_name="s",
        num_cores=NC, num_subcores=NS)

    @functools.partial(
        pl.kernel,
        out_type=jax.ShapeDtypeStruct((total, DS), jnp.float32),
        mesh=mesh,
        scratch_types=[
            pltpu.VMEM((chunk,), jnp.int32),
            pltpu.VMEM((chunk, DS), jnp.float32),
            pltpu.SemaphoreType.DMA,
        ],
        compiler_params=pltpu.CompilerParams(use_tc_tiling_on_sc=False),
    )
    def decode(table_hbm, idx_hbm, out_hbm, idx_v, rows_v, sem):
        wid = lax.axis_index("s") * NC + lax.axis_index("c")
        base = wid * b_per_w
        for c in range(n_chunks):
            off = base + c * chunk
            pltpu.sync_copy(idx_hbm.at[pl.ds(off, chunk)], idx_v)
            pltpu.async_copy(table_hbm.at[idx_v], rows_v, sem).wait()
            pltpu.sync_copy(rows_v, out_hbm.at[pl.ds(off, chunk)])

    return decode


def kernel(vecs, codewords):
    n, d = vecs.shape
    m_, ks_, ds_ = codewords.shape
    cwt = codewords.transpose(0, 2, 1).reshape(m_ * ds_, ks_)
    codes = _encode(vecs, cwt, block_b=1024)
    flat_codes = codes.reshape(n * m_)
    table = codewords.reshape(m_ * ks_, ds_)
    rows = _make_decode(n * m_, 2048)(table, flat_codes)
    return rows.reshape(n, d)
